# speedup vs baseline: 1.9848x; 1.9848x over previous
import functools

import jax
import jax.numpy as jnp
from jax import lax
from jax.experimental import pallas as pl
from jax.experimental.pallas import tpu as pltpu
from jax.experimental.pallas import tpu_sc as plsc

D_MODEL = 1024
NUM_WORKERS = 32
CHUNK = 32


def _make_gather(batch):
    rows_per_worker = batch // NUM_WORKERS
    num_chunks = rows_per_worker // CHUNK
    mesh = plsc.VectorSubcoreMesh(core_axis_name="c", subcore_axis_name="s")

    @functools.partial(
        pl.kernel,
        mesh=mesh,
        out_type=jax.ShapeDtypeStruct((batch, D_MODEL), jnp.float32),
        scratch_types=[
            pltpu.VMEM((rows_per_worker,), jnp.int32),
            pltpu.VMEM((CHUNK, D_MODEL), jnp.float32),
            pltpu.SemaphoreType.DMA,
        ],
    )
    def gather_kernel(table_hbm, idx_hbm, out_hbm, idx_v, rows_v, sem):
        wid = lax.axis_index("s") * 2 + lax.axis_index("c")
        base = wid * rows_per_worker
        pltpu.sync_copy(idx_hbm.at[pl.ds(base, rows_per_worker)], idx_v)

        def body(i, carry):
            off = i * CHUNK
            pltpu.async_copy(
                table_hbm.at[idx_v.at[pl.ds(off, CHUNK)]], rows_v, sem
            ).wait()
            pltpu.sync_copy(rows_v, out_hbm.at[pl.ds(base + off, CHUNK)])
            return carry

        lax.fori_loop(0, num_chunks, body, 0)

    return gather_kernel


def kernel(positions, pe):
    b, s = positions.shape
    flat = positions.reshape(b * s)
    out = _make_gather(b * s)(pe, flat)
    return out.reshape(b, s, pe.shape[1])


# speedup vs baseline: 2.3189x; 1.1683x over previous
import functools

import jax
import jax.numpy as jnp
from jax import lax
from jax.experimental import pallas as pl
from jax.experimental.pallas import tpu as pltpu
from jax.experimental.pallas import tpu_sc as plsc

D_MODEL = 1024
NUM_WORKERS = 32
CHUNK = 16
NBUF = 4


def _make_gather(batch):
    rows_per_worker = batch // NUM_WORKERS
    num_chunks = rows_per_worker // CHUNK
    num_groups = num_chunks // NBUF
    mesh = plsc.VectorSubcoreMesh(core_axis_name="c", subcore_axis_name="s")

    @functools.partial(
        pl.kernel,
        mesh=mesh,
        out_type=jax.ShapeDtypeStruct((batch, D_MODEL), jnp.float32),
        scratch_types=[
            pltpu.VMEM((rows_per_worker,), jnp.int32),
        ]
        + [pltpu.VMEM((CHUNK, D_MODEL), jnp.float32) for _ in range(NBUF)]
        + [pltpu.SemaphoreType.DMA for _ in range(2 * NBUF)],
    )
    def gather_kernel(table_hbm, idx_hbm, out_hbm, idx_v, *rest):
        bufs = rest[:NBUF]
        gsems = rest[NBUF:2 * NBUF]
        ssems = rest[2 * NBUF:]
        wid = lax.axis_index("s") * 2 + lax.axis_index("c")
        base = wid * rows_per_worker
        pltpu.sync_copy(idx_hbm.at[pl.ds(base, rows_per_worker)], idx_v)

        for b in range(NBUF):
            pltpu.async_copy(
                table_hbm.at[idx_v.at[pl.ds(b * CHUNK, CHUNK)]], bufs[b], gsems[b]
            )

        def group(g, carry):
            goff = g * (NBUF * CHUNK)
            for b in range(NBUF):
                off = goff + b * CHUNK
                pltpu.make_async_copy(
                    table_hbm.at[idx_v.at[pl.ds(off, CHUNK)]], bufs[b], gsems[b]
                ).wait()
                pltpu.async_copy(
                    bufs[b], out_hbm.at[pl.ds(base + off, CHUNK)], ssems[b]
                )
            for b in range(NBUF):
                off = goff + b * CHUNK
                nxt = off + NBUF * CHUNK

                @pl.when(g + 1 < num_groups)
                def _():
                    pltpu.make_async_copy(
                        bufs[b], out_hbm.at[pl.ds(base + off, CHUNK)], ssems[b]
                    ).wait()
                    pltpu.async_copy(
                        table_hbm.at[idx_v.at[pl.ds(nxt, CHUNK)]], bufs[b], gsems[b]
                    )

            return carry

        lax.fori_loop(0, num_groups, group, 0)

        last = (num_groups - 1) * (NBUF * CHUNK)
        for b in range(NBUF):
            off = last + b * CHUNK
            pltpu.make_async_copy(
                bufs[b], out_hbm.at[pl.ds(base + off, CHUNK)], ssems[b]
            ).wait()

    return gather_kernel


def kernel(positions, pe):
    b, s = positions.shape
    flat = positions.reshape(b * s)
    out = _make_gather(b * s)(pe, flat)
    return out.reshape(b, s, pe.shape[1])


# speedup vs baseline: 2.3399x; 1.0091x over previous
import functools

import jax
import jax.numpy as jnp
from jax import lax
from jax.experimental import pallas as pl
from jax.experimental.pallas import tpu as pltpu
from jax.experimental.pallas import tpu_sc as plsc

D_MODEL = 1024
NUM_WORKERS = 32
CHUNK = 8
NBUF = 8


def _make_gather(batch):
    rows_per_worker = batch // NUM_WORKERS
    num_chunks = rows_per_worker // CHUNK
    num_groups = num_chunks // NBUF
    mesh = plsc.VectorSubcoreMesh(core_axis_name="c", subcore_axis_name="s")

    @functools.partial(
        pl.kernel,
        mesh=mesh,
        out_type=jax.ShapeDtypeStruct((batch, D_MODEL), jnp.float32),
        scratch_types=[
            pltpu.VMEM((rows_per_worker,), jnp.int32),
        ]
        + [pltpu.VMEM((CHUNK, D_MODEL), jnp.float32) for _ in range(NBUF)]
        + [pltpu.SemaphoreType.DMA for _ in range(2 * NBUF)],
    )
    def gather_kernel(table_hbm, idx_hbm, out_hbm, idx_v, *rest):
        bufs = rest[:NBUF]
        gsems = rest[NBUF:2 * NBUF]
        ssems = rest[2 * NBUF:]
        wid = lax.axis_index("s") * 2 + lax.axis_index("c")
        base = wid * rows_per_worker
        pltpu.sync_copy(idx_hbm.at[pl.ds(base, rows_per_worker)], idx_v)

        for b in range(NBUF):
            pltpu.async_copy(
                table_hbm.at[idx_v.at[pl.ds(b * CHUNK, CHUNK)]], bufs[b], gsems[b]
            )

        def group(g, carry):
            goff = g * (NBUF * CHUNK)
            for b in range(NBUF):
                off = goff + b * CHUNK
                pltpu.make_async_copy(
                    table_hbm.at[idx_v.at[pl.ds(off, CHUNK)]], bufs[b], gsems[b]
                ).wait()
                pltpu.async_copy(
                    bufs[b], out_hbm.at[pl.ds(base + off, CHUNK)], ssems[b]
                )
            for b in range(NBUF):
                off = goff + b * CHUNK
                nxt = off + NBUF * CHUNK

                @pl.when(g + 1 < num_groups)
                def _():
                    pltpu.make_async_copy(
                        bufs[b], out_hbm.at[pl.ds(base + off, CHUNK)], ssems[b]
                    ).wait()
                    pltpu.async_copy(
                        table_hbm.at[idx_v.at[pl.ds(nxt, CHUNK)]], bufs[b], gsems[b]
                    )

            return carry

        lax.fori_loop(0, num_groups, group, 0)

        last = (num_groups - 1) * (NBUF * CHUNK)
        for b in range(NBUF):
            off = last + b * CHUNK
            pltpu.make_async_copy(
                bufs[b], out_hbm.at[pl.ds(base + off, CHUNK)], ssems[b]
            ).wait()

    return gather_kernel


def kernel(positions, pe):
    b, s = positions.shape
    flat = positions.reshape(b * s)
    out = _make_gather(b * s)(pe, flat)
    return out.reshape(b, s, pe.shape[1])
